# baseline (speedup 1.0000x reference)
import functools

import jax
import jax.numpy as jnp
from jax import lax
from jax.experimental import pallas as pl
from jax.experimental.pallas import tpu as pltpu
from jax.experimental.pallas import tpu_sc as plsc

BATCH = 4096
DIM = 32
HALF = 16

_info = plsc.get_sparse_core_info()
_NC = _info.num_cores
_NS = _info.num_subcores
_NW = _NC * _NS
_BPW = BATCH // _NW

_mesh = plsc.VectorSubcoreMesh(core_axis_name="c", subcore_axis_name="s")


@functools.partial(
    pl.kernel,
    mesh=_mesh,
    compiler_params=pltpu.CompilerParams(use_tc_tiling_on_sc=False),
    out_type=jax.ShapeDtypeStruct((BATCH, DIM), jnp.float32),
    scratch_types=[
        pltpu.VMEM((_BPW,), jnp.int32),
        pltpu.VMEM((_BPW, DIM), jnp.float32),
        pltpu.VMEM((_BPW, DIM), jnp.float32),
        pltpu.VMEM((_BPW, DIM), jnp.float32),
        pltpu.VMEM((DIM,), jnp.float32),
        pltpu.VMEM((DIM,), jnp.float32),
        pltpu.VMEM((DIM,), jnp.float32),
        pltpu.SemaphoreType.DMA,
        pltpu.SemaphoreType.DMA,
    ],
)
def _sc_encoder(ents_h, item_h, ent_h, wve_h, wee_h, be_h, out_h,
                idx_v, v_v, e_v, o_v, wve_v, wee_v, be_v, sem_v, sem_e):
    wid = lax.axis_index("s") * _NC + lax.axis_index("c")
    base = wid * _BPW

    pltpu.sync_copy(ents_h.at[pl.ds(base, _BPW)], idx_v)
    cp_v = pltpu.async_copy(item_h.at[idx_v], v_v, sem_v)
    cp_e = pltpu.async_copy(ent_h.at[idx_v], e_v, sem_e)

    pltpu.sync_copy(wve_h, wve_v)
    pltpu.sync_copy(wee_h, wee_v)
    pltpu.sync_copy(be_h, be_v)

    wve0 = wve_v[pl.ds(0, HALF)]
    wve1 = wve_v[pl.ds(HALF, HALF)]
    wee0 = wee_v[pl.ds(0, HALF)]
    wee1 = wee_v[pl.ds(HALF, HALF)]
    be0 = be_v[pl.ds(0, HALF)]
    be1 = be_v[pl.ds(HALF, HALF)]

    cp_v.wait()
    cp_e.wait()

    lanes = lax.iota(jnp.int32, HALF)
    perms = [lanes ^ (1 << k) for k in range(4)]
    _dnums = lax.GatherDimensionNumbers(
        offset_dims=(), collapsed_slice_dims=(0,), start_index_map=(0,))

    def allsum(t):
        for p in perms:
            t = t + lax.gather(t, p[:, None], _dnums, (1,),
                               mode=lax.GatherScatterMode.PROMISE_IN_BOUNDS)
        return t

    def row(b, carry):
        v0 = v_v[b, pl.ds(0, HALF)]
        v1 = v_v[b, pl.ds(HALF, HALF)]
        e0 = e_v[b, pl.ds(0, HALF)]
        e1 = e_v[b, pl.ds(HALF, HALF)]
        s1 = allsum(e0 * wve0 + e1 * wve1)
        s2 = allsum(v0 * wee0 + v1 * wee1)
        o_v[b, pl.ds(0, HALF)] = v0 * s1 + e0 * s2 + be0
        o_v[b, pl.ds(HALF, HALF)] = v1 * s1 + e1 * s2 + be1
        return carry

    lax.fori_loop(0, _BPW, row, 0)

    pltpu.sync_copy(o_v, out_h.at[pl.ds(base, _BPW)])


def kernel(entities, item_table, ent_table, w_vv, w_ev, w_ve, w_ee, b_v, b_e):
    del w_vv, w_ev, b_v
    idx = entities.astype(jnp.int32)
    return _sc_encoder(idx, item_table, ent_table, w_ve, w_ee, b_e)


# speedup vs baseline: 9.7623x; 9.7623x over previous
import functools

import jax
import jax.numpy as jnp
from jax import lax
from jax.experimental import pallas as pl
from jax.experimental.pallas import tpu as pltpu
from jax.experimental.pallas import tpu_sc as plsc

BATCH = 4096
DIM = 32
HALF = 16
LANE = 128

_info = plsc.get_sparse_core_info()
_NC = _info.num_cores
_NS = _info.num_subcores
_NW = _NC * _NS
_BPW = BATCH // _NW
_CHUNK = 4
_NCH = _BPW // _CHUNK
_SLOTS = 2 * _CHUNK

_mesh = plsc.VectorSubcoreMesh(core_axis_name="c", subcore_axis_name="s")


@functools.partial(
    pl.kernel,
    mesh=_mesh,
    compiler_params=pltpu.CompilerParams(needs_layout_passes=False),
    out_type=jax.ShapeDtypeStruct((DIM, BATCH), jnp.float32),
    scratch_types=[
        pltpu.VMEM((_BPW,), jnp.int32),
        pltpu.VMEM((DIM,), jnp.float32),
        pltpu.VMEM((DIM,), jnp.float32),
        pltpu.VMEM((DIM,), jnp.float32),
        pltpu.VMEM((_SLOTS, DIM, LANE), jnp.float32),
        pltpu.VMEM((_SLOTS, DIM, LANE), jnp.float32),
        pltpu.VMEM((DIM, _BPW), jnp.float32),
        pltpu.SemaphoreType.DMA,
        pltpu.SemaphoreType.DMA,
    ],
)
def _sc_encoder(ents_h, item_h, ent_h, wve_h, wee_h, be_h, out_h,
                idx_v, wve_v, wee_v, be_v, vbuf, ebuf, o_v, sem0, sem1):
    wid = lax.axis_index("s") * _NC + lax.axis_index("c")
    base = wid * _BPW

    pltpu.sync_copy(ents_h.at[pl.ds(base, _BPW)], idx_v)

    def idx_at(j):
        ivec = idx_v[pl.ds((j // HALF) * HALF, HALF)]
        return ivec[j % HALF]
    pltpu.sync_copy(wve_h, wve_v)
    pltpu.sync_copy(wee_h, wee_v)
    pltpu.sync_copy(be_h, be_v)

    wve0 = wve_v[pl.ds(0, HALF)]
    wve1 = wve_v[pl.ds(HALF, HALF)]
    wee0 = wee_v[pl.ds(0, HALF)]
    wee1 = wee_v[pl.ds(HALF, HALF)]
    be0 = be_v[pl.ds(0, HALF)]
    be1 = be_v[pl.ds(HALF, HALF)]

    lanes = lax.iota(jnp.int32, HALF)
    perms = [lanes ^ (1 << k) for k in range(4)]
    _dnums = lax.GatherDimensionNumbers(
        offset_dims=(), collapsed_slice_dims=(0,), start_index_map=(0,))

    def allsum(t):
        for p in perms:
            t = t + lax.gather(t, p[:, None], _dnums, (1,),
                               mode=lax.GatherScatterMode.PROMISE_IN_BOUNDS)
        return t

    sems = (sem0, sem1)

    def issue(c):
        cps = []
        sem = sems[c % 2]
        for j in range(c * _CHUNK, (c + 1) * _CHUNK):
            idx = idx_at(j)
            col = idx & (LANE - 1)
            cb = pl.multiple_of(idx - col, LANE)
            slot = j % _SLOTS
            cps.append(pltpu.async_copy(
                item_h.at[:, pl.ds(cb, LANE)], vbuf.at[slot], sem))
            cps.append(pltpu.async_copy(
                ent_h.at[:, pl.ds(cb, LANE)], ebuf.at[slot], sem))
        return cps

    inflight = {0: issue(0)}
    for c in range(_NCH):
        if c + 1 < _NCH:
            inflight[c + 1] = issue(c + 1)
        for cp in inflight.pop(c):
            cp.wait()
        for j in range(c * _CHUNK, (c + 1) * _CHUNK):
            idx = idx_at(j)
            colv = jnp.full((HALF,), idx & (LANE - 1), jnp.int32)
            slot = j % _SLOTS
            d_lo = lanes
            d_hi = lanes + HALF
            v0 = plsc.load_gather(vbuf.at[slot], [d_lo, colv])
            v1 = plsc.load_gather(vbuf.at[slot], [d_hi, colv])
            e0 = plsc.load_gather(ebuf.at[slot], [d_lo, colv])
            e1 = plsc.load_gather(ebuf.at[slot], [d_hi, colv])
            s1 = allsum(e0 * wve0 + e1 * wve1)
            s2 = allsum(v0 * wee0 + v1 * wee1)
            jv = jnp.full((HALF,), j, jnp.int32)
            plsc.store_scatter(o_v, [d_lo, jv], v0 * s1 + e0 * s2 + be0)
            plsc.store_scatter(o_v, [d_hi, jv], v1 * s1 + e1 * s2 + be1)

    pltpu.sync_copy(o_v, out_h.at[:, pl.ds(base, _BPW)])


def kernel(entities, item_table, ent_table, w_vv, w_ev, w_ve, w_ee, b_v, b_e):
    del w_vv, w_ev, b_v
    idx = entities.astype(jnp.int32)
    out_t = _sc_encoder(idx, item_table.T, ent_table.T, w_ve, w_ee, b_e)
    return out_t.T


# speedup vs baseline: 9.8024x; 1.0041x over previous
import functools

import jax
import jax.numpy as jnp
from jax import lax
from jax.experimental import pallas as pl
from jax.experimental.pallas import tpu as pltpu
from jax.experimental.pallas import tpu_sc as plsc

BATCH = 4096
DIM = 32
HALF = 16
LANE = 128

_info = plsc.get_sparse_core_info()
_NC = _info.num_cores
_NS = _info.num_subcores
_NW = _NC * _NS
_BPW = BATCH // _NW
_CHUNK = 4
_NCH = _BPW // _CHUNK
_SLOTS = 2 * _CHUNK

_mesh = plsc.VectorSubcoreMesh(core_axis_name="c", subcore_axis_name="s")


@functools.partial(
    pl.kernel,
    mesh=_mesh,
    compiler_params=pltpu.CompilerParams(needs_layout_passes=False),
    out_type=jax.ShapeDtypeStruct((DIM, BATCH), jnp.float32),
    scratch_types=[
        pltpu.VMEM((_BPW,), jnp.int32),
        pltpu.VMEM((DIM,), jnp.float32),
        pltpu.VMEM((DIM,), jnp.float32),
        pltpu.VMEM((DIM,), jnp.float32),
        pltpu.VMEM((_SLOTS, DIM, LANE), jnp.float32),
        pltpu.VMEM((_SLOTS, DIM, LANE), jnp.float32),
        pltpu.VMEM((DIM, _BPW), jnp.float32),
        pltpu.SemaphoreType.DMA,
        pltpu.SemaphoreType.DMA,
    ],
)
def _sc_encoder(ents_h, item_h, ent_h, wve_h, wee_h, be_h, out_h,
                idx_v, wve_v, wee_v, be_v, vbuf, ebuf, o_v, sem0, sem1):
    wid = lax.axis_index("s") * _NC + lax.axis_index("c")
    base = wid * _BPW

    pltpu.sync_copy(ents_h.at[pl.ds(base, _BPW)], idx_v)

    def idx_at(j):
        ivec = idx_v[pl.ds((j // HALF) * HALF, HALF)]
        return ivec[j % HALF]

    sems = (sem0, sem1)

    def issue(c):
        cps = []
        sem = sems[c % 2]
        for j in range(c * _CHUNK, (c + 1) * _CHUNK):
            idx = idx_at(j)
            col = idx & (LANE - 1)
            cb = pl.multiple_of(idx - col, LANE)
            slot = j % _SLOTS
            cps.append(pltpu.async_copy(
                item_h.at[:, pl.ds(cb, LANE)], vbuf.at[slot], sem))
            cps.append(pltpu.async_copy(
                ent_h.at[:, pl.ds(cb, LANE)], ebuf.at[slot], sem))
        return cps

    inflight = {0: issue(0)}

    pltpu.sync_copy(wve_h, wve_v)
    pltpu.sync_copy(wee_h, wee_v)
    pltpu.sync_copy(be_h, be_v)

    wve0 = wve_v[pl.ds(0, HALF)]
    wve1 = wve_v[pl.ds(HALF, HALF)]
    wee0 = wee_v[pl.ds(0, HALF)]
    wee1 = wee_v[pl.ds(HALF, HALF)]
    be0 = be_v[pl.ds(0, HALF)]
    be1 = be_v[pl.ds(HALF, HALF)]

    lanes = lax.iota(jnp.int32, HALF)
    perms = [lanes ^ (1 << k) for k in range(4)]
    _dnums = lax.GatherDimensionNumbers(
        offset_dims=(), collapsed_slice_dims=(0,), start_index_map=(0,))

    def allsum(t):
        for p in perms:
            t = t + lax.gather(t, p[:, None], _dnums, (1,),
                               mode=lax.GatherScatterMode.PROMISE_IN_BOUNDS)
        return t

    d_lo = lanes
    d_hi = lanes + HALF
    for c in range(_NCH):
        if c + 1 < _NCH:
            inflight[c + 1] = issue(c + 1)
        for cp in inflight.pop(c):
            cp.wait()
        for j in range(c * _CHUNK, (c + 1) * _CHUNK):
            idx = idx_at(j)
            colv = jnp.full((HALF,), idx & (LANE - 1), jnp.int32)
            slot = j % _SLOTS
            v0 = plsc.load_gather(vbuf.at[slot], [d_lo, colv])
            v1 = plsc.load_gather(vbuf.at[slot], [d_hi, colv])
            e0 = plsc.load_gather(ebuf.at[slot], [d_lo, colv])
            e1 = plsc.load_gather(ebuf.at[slot], [d_hi, colv])
            s1 = allsum(e0 * wve0 + e1 * wve1)
            s2 = allsum(v0 * wee0 + v1 * wee1)
            jv = jnp.full((HALF,), j, jnp.int32)
            plsc.store_scatter(o_v, [d_lo, jv], v0 * s1 + e0 * s2 + be0)
            plsc.store_scatter(o_v, [d_hi, jv], v1 * s1 + e1 * s2 + be1)

    pltpu.sync_copy(o_v, out_h.at[:, pl.ds(base, _BPW)])


def kernel(entities, item_table, ent_table, w_vv, w_ev, w_ve, w_ee, b_v, b_e):
    del w_vv, w_ev, b_v
    idx = entities.astype(jnp.int32)
    out_t = _sc_encoder(idx, item_table.T, ent_table.T, w_ve, w_ee, b_e)
    return out_t.T


# speedup vs baseline: 9.8194x; 1.0017x over previous
import functools

import jax
import jax.numpy as jnp
from jax import lax
from jax.experimental import pallas as pl
from jax.experimental.pallas import tpu as pltpu
from jax.experimental.pallas import tpu_sc as plsc

BATCH = 4096
DIM = 32
HALF = 16
LANE = 128

_info = plsc.get_sparse_core_info()
_NC = _info.num_cores
_NS = _info.num_subcores
_NW = _NC * _NS
_BPW = BATCH // _NW
_CHUNK = 4
_NCH = _BPW // _CHUNK
_SLOTS = 2 * _CHUNK

_mesh = plsc.VectorSubcoreMesh(core_axis_name="c", subcore_axis_name="s")


@functools.partial(
    pl.kernel,
    mesh=_mesh,
    compiler_params=pltpu.CompilerParams(needs_layout_passes=False),
    out_type=jax.ShapeDtypeStruct((DIM, BATCH), jnp.float32),
    scratch_types=[
        pltpu.VMEM((_BPW,), jnp.int32),
        pltpu.VMEM((DIM,), jnp.float32),
        pltpu.VMEM((DIM,), jnp.float32),
        pltpu.VMEM((DIM,), jnp.float32),
        pltpu.VMEM((_SLOTS, DIM, LANE), jnp.float32),
        pltpu.VMEM((_SLOTS, DIM, LANE), jnp.float32),
        pltpu.VMEM((_BPW, DIM), jnp.float32),
        pltpu.VMEM((_BPW, DIM), jnp.float32),
        pltpu.VMEM((DIM, _BPW), jnp.float32),
        pltpu.SemaphoreType.DMA,
        pltpu.SemaphoreType.DMA,
    ],
)
def _sc_encoder(ents_h, item_h, ent_h, wve_h, wee_h, be_h, out_h,
                idx_v, wve_v, wee_v, be_v, vbuf, ebuf, v_x, e_x, o_v,
                sem0, sem1):
    wid = lax.axis_index("s") * _NC + lax.axis_index("c")
    base = wid * _BPW

    pltpu.sync_copy(ents_h.at[pl.ds(base, _BPW)], idx_v)

    def idx_at(j):
        ivec = idx_v[pl.ds((j // HALF) * HALF, HALF)]
        return ivec[j % HALF]

    sems = (sem0, sem1)

    def issue(c):
        cps = []
        sem = sems[c % 2]
        for j in range(c * _CHUNK, (c + 1) * _CHUNK):
            idx = idx_at(j)
            col = idx & (LANE - 1)
            cb = pl.multiple_of(idx - col, LANE)
            slot = j % _SLOTS
            cps.append(pltpu.async_copy(
                item_h.at[:, pl.ds(cb, LANE)], vbuf.at[slot], sem))
            cps.append(pltpu.async_copy(
                ent_h.at[:, pl.ds(cb, LANE)], ebuf.at[slot], sem))
        return cps

    inflight = {0: issue(0)}

    pltpu.sync_copy(wve_h, wve_v)
    pltpu.sync_copy(wee_h, wee_v)
    pltpu.sync_copy(be_h, be_v)

    wve0 = wve_v[pl.ds(0, HALF)]
    wve1 = wve_v[pl.ds(HALF, HALF)]
    wee0 = wee_v[pl.ds(0, HALF)]
    wee1 = wee_v[pl.ds(HALF, HALF)]
    be0 = be_v[pl.ds(0, HALF)]
    be1 = be_v[pl.ds(HALF, HALF)]

    lanes = lax.iota(jnp.int32, HALF)
    perms = [lanes ^ (1 << k) for k in range(4)]
    _dnums = lax.GatherDimensionNumbers(
        offset_dims=(), collapsed_slice_dims=(0,), start_index_map=(0,))

    def allsum(t):
        for p in perms:
            t = t + lax.gather(t, p[:, None], _dnums, (1,),
                               mode=lax.GatherScatterMode.PROMISE_IN_BOUNDS)
        return t

    d_lo = lanes
    d_hi = lanes + HALF
    for c in range(_NCH):
        if c + 1 < _NCH:
            inflight[c + 1] = issue(c + 1)
        for cp in inflight.pop(c):
            cp.wait()
        for j in range(c * _CHUNK, (c + 1) * _CHUNK):
            idx = idx_at(j)
            colv = jnp.full((HALF,), idx & (LANE - 1), jnp.int32)
            slot = j % _SLOTS
            v_x[j, pl.ds(0, HALF)] = plsc.load_gather(vbuf.at[slot], [d_lo, colv])
            v_x[j, pl.ds(HALF, HALF)] = plsc.load_gather(vbuf.at[slot], [d_hi, colv])
            e_x[j, pl.ds(0, HALF)] = plsc.load_gather(ebuf.at[slot], [d_lo, colv])
            e_x[j, pl.ds(HALF, HALF)] = plsc.load_gather(ebuf.at[slot], [d_hi, colv])

    def row(j, carry):
        v0 = v_x[j, pl.ds(0, HALF)]
        v1 = v_x[j, pl.ds(HALF, HALF)]
        e0 = e_x[j, pl.ds(0, HALF)]
        e1 = e_x[j, pl.ds(HALF, HALF)]
        s1 = allsum(e0 * wve0 + e1 * wve1)
        s2 = allsum(v0 * wee0 + v1 * wee1)
        jv = jnp.full((HALF,), j, jnp.int32)
        plsc.store_scatter(o_v, [d_lo, jv], v0 * s1 + e0 * s2 + be0)
        plsc.store_scatter(o_v, [d_hi, jv], v1 * s1 + e1 * s2 + be1)
        return carry

    lax.fori_loop(0, _BPW, row, 0)

    pltpu.sync_copy(o_v, out_h.at[:, pl.ds(base, _BPW)])


def kernel(entities, item_table, ent_table, w_vv, w_ev, w_ve, w_ee, b_v, b_e):
    del w_vv, w_ev, b_v
    idx = entities.astype(jnp.int32)
    out_t = _sc_encoder(idx, item_table.T, ent_table.T, w_ve, w_ee, b_e)
    return out_t.T


# speedup vs baseline: 9.8488x; 1.0030x over previous
import functools

import jax
import jax.numpy as jnp
from jax import lax
from jax.experimental import pallas as pl
from jax.experimental.pallas import tpu as pltpu
from jax.experimental.pallas import tpu_sc as plsc

BATCH = 4096
DIM = 32
HALF = 16
LANE = 128

_info = plsc.get_sparse_core_info()
_NC = _info.num_cores
_NS = _info.num_subcores
_NW = _NC * _NS
_BPW = BATCH // _NW
_CHUNK = 4
_NCH = _BPW // _CHUNK
_SLOTS = 2 * _CHUNK

_mesh = plsc.VectorSubcoreMesh(core_axis_name="c", subcore_axis_name="s")


@functools.partial(
    pl.kernel,
    mesh=_mesh,
    compiler_params=pltpu.CompilerParams(needs_layout_passes=False),
    out_type=jax.ShapeDtypeStruct((DIM, BATCH), jnp.float32),
    scratch_types=[
        pltpu.VMEM((_BPW,), jnp.int32),
        pltpu.VMEM((DIM,), jnp.float32),
        pltpu.VMEM((DIM,), jnp.float32),
        pltpu.VMEM((DIM,), jnp.float32),
        pltpu.VMEM((_SLOTS, DIM, LANE), jnp.float32),
        pltpu.VMEM((_SLOTS, DIM, LANE), jnp.float32),
        pltpu.VMEM((_BPW, DIM), jnp.float32),
        pltpu.VMEM((_BPW, DIM), jnp.float32),
        pltpu.VMEM((DIM, _BPW), jnp.float32),
        pltpu.SemaphoreType.DMA,
        pltpu.SemaphoreType.DMA,
    ],
)
def _sc_encoder(ents_h, item_h, ent_h, wve_h, wee_h, be_h, out_h,
                idx_v, wve_v, wee_v, be_v, vbuf, ebuf, v_x, e_x, o_v,
                sem0, sem1):
    wid = lax.axis_index("s") * _NC + lax.axis_index("c")
    base = wid * _BPW

    pltpu.sync_copy(ents_h.at[pl.ds(base, _BPW)], idx_v)

    def idx_at(j):
        ivec = idx_v[pl.ds((j // HALF) * HALF, HALF)]
        return ivec[j % HALF]

    sems = (sem0, sem1)

    def issue(c):
        cps = []
        sem = sems[c % 2]
        for j in range(c * _CHUNK, (c + 1) * _CHUNK):
            idx = idx_at(j)
            col = idx & (LANE - 1)
            cb = pl.multiple_of(idx - col, LANE)
            slot = j % _SLOTS
            cps.append(pltpu.async_copy(
                item_h.at[:, pl.ds(cb, LANE)], vbuf.at[slot], sem))
            cps.append(pltpu.async_copy(
                ent_h.at[:, pl.ds(cb, LANE)], ebuf.at[slot], sem))
        return cps

    inflight = {0: issue(0)}

    pltpu.sync_copy(wve_h, wve_v)
    pltpu.sync_copy(wee_h, wee_v)
    pltpu.sync_copy(be_h, be_v)

    wve0 = wve_v[pl.ds(0, HALF)]
    wve1 = wve_v[pl.ds(HALF, HALF)]
    wee0 = wee_v[pl.ds(0, HALF)]
    wee1 = wee_v[pl.ds(HALF, HALF)]
    be0 = be_v[pl.ds(0, HALF)]
    be1 = be_v[pl.ds(HALF, HALF)]

    lanes = lax.iota(jnp.int32, HALF)
    perms = [lanes ^ (1 << k) for k in range(4)]
    _dnums = lax.GatherDimensionNumbers(
        offset_dims=(), collapsed_slice_dims=(0,), start_index_map=(0,))

    def allsum(t):
        for p in perms:
            t = t + lax.gather(t, p[:, None], _dnums, (1,),
                               mode=lax.GatherScatterMode.PROMISE_IN_BOUNDS)
        return t

    d_lo = lanes
    d_hi = lanes + HALF
    inflight[1] = issue(1)
    for c in range(_NCH):
        for cp in inflight.pop(c):
            cp.wait()
        for j in range(c * _CHUNK, (c + 1) * _CHUNK):
            idx = idx_at(j)
            colv = jnp.full((HALF,), idx & (LANE - 1), jnp.int32)
            slot = j % _SLOTS
            v_x[j, pl.ds(0, HALF)] = plsc.load_gather(vbuf.at[slot], [d_lo, colv])
            v_x[j, pl.ds(HALF, HALF)] = plsc.load_gather(vbuf.at[slot], [d_hi, colv])
            e_x[j, pl.ds(0, HALF)] = plsc.load_gather(ebuf.at[slot], [d_lo, colv])
            e_x[j, pl.ds(HALF, HALF)] = plsc.load_gather(ebuf.at[slot], [d_hi, colv])
        if c + 2 < _NCH:
            inflight[c + 2] = issue(c + 2)

    def row(j, carry):
        v0 = v_x[j, pl.ds(0, HALF)]
        v1 = v_x[j, pl.ds(HALF, HALF)]
        e0 = e_x[j, pl.ds(0, HALF)]
        e1 = e_x[j, pl.ds(HALF, HALF)]
        s1 = allsum(e0 * wve0 + e1 * wve1)
        s2 = allsum(v0 * wee0 + v1 * wee1)
        jv = jnp.full((HALF,), j, jnp.int32)
        plsc.store_scatter(o_v, [d_lo, jv], v0 * s1 + e0 * s2 + be0)
        plsc.store_scatter(o_v, [d_hi, jv], v1 * s1 + e1 * s2 + be1)
        return carry

    lax.fori_loop(0, _BPW, row, 0)

    pltpu.sync_copy(o_v, out_h.at[:, pl.ds(base, _BPW)])


def kernel(entities, item_table, ent_table, w_vv, w_ev, w_ve, w_ee, b_v, b_e):
    del w_vv, w_ev, b_v
    idx = entities.astype(jnp.int32)
    out_t = _sc_encoder(idx, item_table.T, ent_table.T, w_ve, w_ee, b_e)
    return out_t.T
